# Initial kernel scaffold; baseline (speedup 1.0000x reference)
import jax
import jax.numpy as jnp
from jax.experimental import pallas as pl


def kernel(node_emb, r_emb_w, r_proj_w, edge_index_t, edge_attr):
    raise NotImplementedError("write your pallas kernel here")


# speedup vs baseline: 3.1154x; 3.1154x over previous
import functools

import jax
import jax.numpy as jnp
from jax import lax
from jax.experimental import pallas as pl
from jax.experimental.pallas import tpu as pltpu
from jax.experimental.pallas import tpu_sc as plsc

_D = 32
_E = 200000
_NW = 32
_CHUNK = 128
_CH = 49
_EPAD = _NW * _CH * _CHUNK
_BT = 2048
_GB = _EPAD // _BT


def _sc_gather(node_emb, head_idx, tail_idx):
    mesh = plsc.VectorSubcoreMesh(core_axis_name="c", subcore_axis_name="s")

    @functools.partial(
        pl.kernel,
        mesh=mesh,
        out_type=[
            jax.ShapeDtypeStruct((_NW, _CH, _CHUNK, _D), jnp.float32),
            jax.ShapeDtypeStruct((_NW, _CH, _CHUNK, _D), jnp.float32),
        ],
        scratch_types=[
            pltpu.VMEM((_CH, _CHUNK), jnp.int32),
            pltpu.VMEM((_CH, _CHUNK), jnp.int32),
            pltpu.VMEM((_CHUNK, _D), jnp.float32),
            pltpu.VMEM((_CHUNK, _D), jnp.float32),
            pltpu.SemaphoreType.DMA,
            pltpu.SemaphoreType.DMA,
        ],
        compiler_params=pltpu.CompilerParams(use_tc_tiling_on_sc=False),
    )
    def gather_kernel(node_hbm, hidx_hbm, tidx_hbm, hout_hbm, tout_hbm,
                      hidx_v, tidx_v, hbuf, tbuf, sem_h, sem_t):
        wid = lax.axis_index("s") * 2 + lax.axis_index("c")
        pltpu.sync_copy(hidx_hbm.at[wid], hidx_v)
        pltpu.sync_copy(tidx_hbm.at[wid], tidx_v)

        def body(c, carry):
            cp_h = pltpu.async_copy(node_hbm.at[hidx_v.at[c]], hbuf, sem_h)
            cp_t = pltpu.async_copy(node_hbm.at[tidx_v.at[c]], tbuf, sem_t)
            cp_h.wait()
            cp_t.wait()
            pltpu.sync_copy(hbuf, hout_hbm.at[wid, c])
            pltpu.sync_copy(tbuf, tout_hbm.at[wid, c])
            return carry

        lax.fori_loop(0, _CH, body, 0)

    return gather_kernel(node_emb, head_idx, tail_idx)


def _tc_loss(head2d, tail2d, ridx3, p_stacked, r_emb_w):
    def body(h_ref, t_ref, r_ref, p_ref, e_ref, o_ref):
        i = pl.program_id(0)
        ridx = r_ref[0, 0, :]
        diff = h_ref[...] - t_ref[...]
        g = lax.shift_right_logical(ridx, 1)

        trow = lax.broadcasted_iota(jnp.int32, (_D, _D * _D), 0)
        tcol = lax.broadcasted_iota(jnp.int32, (_D, _D * _D), 1)
        tmat = ((tcol & (_D - 1)) == trow).astype(jnp.float32)
        diff_t = jnp.dot(diff, tmat, preferred_element_type=jnp.float32)

        col = lax.broadcasted_iota(jnp.int32, (_BT, _D * _D), 1)
        sel = (lax.shift_right_logical(col, 5) == g[:, None])
        x = jnp.where(sel, diff_t, 0.0)

        out = jnp.dot(x, p_ref[...], preferred_element_type=jnp.float32)

        rcol = lax.broadcasted_iota(jnp.int32, (_BT, 64), 1)
        onehot_r = (rcol == ridx[:, None]).astype(jnp.float32)
        r_e = jnp.dot(onehot_r, e_ref[...], preferred_element_type=jnp.float32)

        s = out + r_e
        row = i * _BT + lax.broadcasted_iota(jnp.int32, (_BT, 1), 0)
        s = jnp.where(row < _E, s, 0.0)
        part = jnp.sum(s * s)

        @pl.when(i == 0)
        def _init():
            o_ref[...] = jnp.zeros((1, 1), jnp.float32)

        o_ref[...] = o_ref[...] + part

        @pl.when(i == _GB - 1)
        def _final():
            o_ref[...] = o_ref[...] * (1.0 / (_E * _D))

    return pl.pallas_call(
        body,
        grid=(_GB,),
        in_specs=[
            pl.BlockSpec((_BT, _D), lambda i: (i, 0)),
            pl.BlockSpec((_BT, _D), lambda i: (i, 0)),
            pl.BlockSpec((1, 1, _BT), lambda i: (i, 0, 0)),
            pl.BlockSpec((_D * _D, _D), lambda i: (0, 0)),
            pl.BlockSpec((64, _D), lambda i: (0, 0)),
        ],
        out_specs=pl.BlockSpec((1, 1), lambda i: (0, 0)),
        out_shape=jax.ShapeDtypeStruct((1, 1), jnp.float32),
    )(head2d, tail2d, ridx3, p_stacked, r_emb_w)


def kernel(node_emb, r_emb_w, r_proj_w, edge_index_t, edge_attr):
    pad = _EPAD - _E
    head_idx = jnp.concatenate(
        [edge_index_t[:, 0], jnp.zeros((pad,), jnp.int32)]).astype(jnp.int32)
    tail_idx = jnp.concatenate(
        [edge_index_t[:, 1], jnp.zeros((pad,), jnp.int32)]).astype(jnp.int32)
    head_idx = head_idx.reshape(_NW, _CH, _CHUNK)
    tail_idx = tail_idx.reshape(_NW, _CH, _CHUNK)

    head_rows, tail_rows = _sc_gather(node_emb, head_idx, tail_idx)

    ridx = jnp.concatenate(
        [edge_attr[:, 0], jnp.zeros((pad,), jnp.int32)]).astype(jnp.int32)
    ridx3 = ridx.reshape(_GB, 1, _BT)

    p_stacked = r_proj_w.reshape(_D * _D, _D)

    loss = _tc_loss(head_rows.reshape(_EPAD, _D),
                    tail_rows.reshape(_EPAD, _D),
                    ridx3, p_stacked, r_emb_w)
    return loss[0, 0]


# speedup vs baseline: 3.2993x; 1.0591x over previous
import functools

import jax
import jax.numpy as jnp
from jax import lax
from jax.experimental import pallas as pl
from jax.experimental.pallas import tpu as pltpu
from jax.experimental.pallas import tpu_sc as plsc

_D = 32
_E = 200000
_NW = 32
_CHUNK = 128
_CH = 49
_EPAD = _NW * _CH * _CHUNK
_BT = 2048
_GB = _EPAD // _BT


def _sc_gather(node_emb, head_idx, tail_idx):
    mesh = plsc.VectorSubcoreMesh(core_axis_name="c", subcore_axis_name="s")

    @functools.partial(
        pl.kernel,
        mesh=mesh,
        out_type=jax.ShapeDtypeStruct((_NW, _CH, _CHUNK, _D), jnp.float32),
        scratch_types=[
            pltpu.VMEM((_CH, _CHUNK), jnp.int32),
            pltpu.VMEM((_CH, _CHUNK), jnp.int32),
            pltpu.VMEM((_CHUNK, _D), jnp.float32),
            pltpu.VMEM((_CHUNK, _D), jnp.float32),
            pltpu.SemaphoreType.DMA,
            pltpu.SemaphoreType.DMA,
        ],
        compiler_params=pltpu.CompilerParams(use_tc_tiling_on_sc=False),
    )
    def gather_kernel(node_hbm, hidx_hbm, tidx_hbm, dout_hbm,
                      hidx_v, tidx_v, hbuf, tbuf, sem_h, sem_t):
        wid = lax.axis_index("s") * 2 + lax.axis_index("c")
        pltpu.sync_copy(hidx_hbm.at[wid], hidx_v)
        pltpu.sync_copy(tidx_hbm.at[wid], tidx_v)

        def body(c, carry):
            cp_h = pltpu.async_copy(node_hbm.at[hidx_v.at[c]], hbuf, sem_h)
            cp_t = pltpu.async_copy(node_hbm.at[tidx_v.at[c]], tbuf, sem_t)
            cp_h.wait()
            cp_t.wait()

            def sub_row(r, carry2):
                hbuf[r, pl.ds(0, 16)] = hbuf[r, pl.ds(0, 16)] - tbuf[r, pl.ds(0, 16)]
                hbuf[r, pl.ds(16, 16)] = hbuf[r, pl.ds(16, 16)] - tbuf[r, pl.ds(16, 16)]
                return carry2

            lax.fori_loop(0, _CHUNK, sub_row, 0)
            pltpu.sync_copy(hbuf, dout_hbm.at[wid, c])
            return carry

        lax.fori_loop(0, _CH, body, 0)

    return gather_kernel(node_emb, head_idx, tail_idx)


def _tc_loss(diff2d, ridx3, p_stacked, r_emb_w):
    def body(d_ref, r_ref, p_ref, e_ref, o_ref):
        i = pl.program_id(0)
        ridx = r_ref[0, 0, :]
        diff = d_ref[...]
        g = lax.shift_right_logical(ridx, 1)

        trow = lax.broadcasted_iota(jnp.int32, (_D, _D * _D), 0)
        tcol = lax.broadcasted_iota(jnp.int32, (_D, _D * _D), 1)
        tmat = ((tcol & (_D - 1)) == trow).astype(jnp.float32)
        diff_t = jnp.dot(diff, tmat, preferred_element_type=jnp.float32)

        col = lax.broadcasted_iota(jnp.int32, (_BT, _D * _D), 1)
        sel = (lax.shift_right_logical(col, 5) == g[:, None])
        x = jnp.where(sel, diff_t, 0.0)

        out = jnp.dot(x, p_ref[...], preferred_element_type=jnp.float32)

        rcol = lax.broadcasted_iota(jnp.int32, (_BT, 64), 1)
        onehot_r = (rcol == ridx[:, None]).astype(jnp.float32)
        r_e = jnp.dot(onehot_r, e_ref[...], preferred_element_type=jnp.float32)

        s = out + r_e
        row = i * _BT + lax.broadcasted_iota(jnp.int32, (_BT, 1), 0)
        s = jnp.where(row < _E, s, 0.0)
        part = jnp.sum(s * s)

        @pl.when(i == 0)
        def _init():
            o_ref[...] = jnp.zeros((1, 1), jnp.float32)

        o_ref[...] = o_ref[...] + part

        @pl.when(i == _GB - 1)
        def _final():
            o_ref[...] = o_ref[...] * (1.0 / (_E * _D))

    return pl.pallas_call(
        body,
        grid=(_GB,),
        in_specs=[
            pl.BlockSpec((_BT, _D), lambda i: (i, 0)),
            pl.BlockSpec((1, 1, _BT), lambda i: (i, 0, 0)),
            pl.BlockSpec((_D * _D, _D), lambda i: (0, 0)),
            pl.BlockSpec((64, _D), lambda i: (0, 0)),
        ],
        out_specs=pl.BlockSpec((1, 1), lambda i: (0, 0)),
        out_shape=jax.ShapeDtypeStruct((1, 1), jnp.float32),
    )(diff2d, ridx3, p_stacked, r_emb_w)


def kernel(node_emb, r_emb_w, r_proj_w, edge_index_t, edge_attr):
    pad = _EPAD - _E
    head_idx = jnp.concatenate(
        [edge_index_t[:, 0], jnp.zeros((pad,), jnp.int32)]).astype(jnp.int32)
    tail_idx = jnp.concatenate(
        [edge_index_t[:, 1], jnp.zeros((pad,), jnp.int32)]).astype(jnp.int32)
    head_idx = head_idx.reshape(_NW, _CH, _CHUNK)
    tail_idx = tail_idx.reshape(_NW, _CH, _CHUNK)

    diff_rows = _sc_gather(node_emb, head_idx, tail_idx)

    ridx = jnp.concatenate(
        [edge_attr[:, 0], jnp.zeros((pad,), jnp.int32)]).astype(jnp.int32)
    ridx3 = ridx.reshape(_GB, 1, _BT)

    p_stacked = r_proj_w.reshape(_D * _D, _D)

    loss = _tc_loss(diff_rows.reshape(_EPAD, _D),
                    ridx3, p_stacked, r_emb_w)
    return loss[0, 0]


# speedup vs baseline: 3.3984x; 1.0300x over previous
import functools

import jax
import jax.numpy as jnp
from jax import lax
from jax.experimental import pallas as pl
from jax.experimental.pallas import tpu as pltpu
from jax.experimental.pallas import tpu_sc as plsc

_D = 32
_E = 200000
_NW = 32
_CHUNK = 128
_CH = 49
_EPAD = _NW * _CH * _CHUNK
_BT = 2048
_GB = _EPAD // _BT


def _sc_gather(node_emb, head_idx, tail_idx):
    mesh = plsc.VectorSubcoreMesh(core_axis_name="c", subcore_axis_name="s")

    @functools.partial(
        pl.kernel,
        mesh=mesh,
        out_type=jax.ShapeDtypeStruct((_NW, _CH, _CHUNK * _D // 128, 128),
                                      jnp.float32),
        scratch_types=[
            pltpu.VMEM((_CH, _CHUNK), jnp.int32),
            pltpu.VMEM((_CH, _CHUNK), jnp.int32),
            pltpu.VMEM((_CHUNK, _D), jnp.float32),
            pltpu.VMEM((_CHUNK, _D), jnp.float32),
            pltpu.VMEM((_CHUNK * _D // 128, 128), jnp.float32),
            pltpu.SemaphoreType.DMA,
            pltpu.SemaphoreType.DMA,
        ],
        compiler_params=pltpu.CompilerParams(use_tc_tiling_on_sc=False),
    )
    def gather_kernel(node_hbm, hidx_hbm, tidx_hbm, dout_hbm,
                      hidx_v, tidx_v, hbuf, tbuf, dbuf, sem_h, sem_t):
        wid = lax.axis_index("s") * 2 + lax.axis_index("c")
        pltpu.sync_copy(hidx_hbm.at[wid], hidx_v)
        pltpu.sync_copy(tidx_hbm.at[wid], tidx_v)

        def body(c, carry):
            cp_h = pltpu.async_copy(node_hbm.at[hidx_v.at[c]], hbuf, sem_h)
            cp_t = pltpu.async_copy(node_hbm.at[tidx_v.at[c]], tbuf, sem_t)
            cp_h.wait()
            cp_t.wait()

            def sub_vec(k, carry2):
                v = (hbuf[lax.shift_right_logical(k, 1), pl.ds((k & 1) * 16, 16)]
                     - tbuf[lax.shift_right_logical(k, 1), pl.ds((k & 1) * 16, 16)])
                dbuf[lax.shift_right_logical(k, 3), pl.ds((k & 7) * 16, 16)] = v
                return carry2

            lax.fori_loop(0, _CHUNK * _D // 16, sub_vec, 0)
            pltpu.sync_copy(dbuf, dout_hbm.at[wid, c])
            return carry

        lax.fori_loop(0, _CH, body, 0)

    return gather_kernel(node_emb, head_idx, tail_idx)


def _tc_loss(diff2d, ridx3, p_stacked, r_emb_w):
    _BR = _BT // 4

    def body(d_ref, r_ref, p_ref, e_ref, o_ref):
        i = pl.program_id(0)
        blk = d_ref[...]

        trow = lax.broadcasted_iota(jnp.int32, (_D, _D * _D), 0)
        tcol = lax.broadcasted_iota(jnp.int32, (_D, _D * _D), 1)
        tmat = ((tcol & (_D - 1)) == trow).astype(jnp.float32)
        col = lax.broadcasted_iota(jnp.int32, (_BR, _D * _D), 1)
        gcol = lax.shift_right_logical(col, 5)
        rcol = lax.broadcasted_iota(jnp.int32, (_BR, 64), 1)
        krow = lax.broadcasted_iota(jnp.int32, (_BR, 1), 0)

        part = jnp.zeros((), jnp.float32)
        for j in range(4):
            dj = blk[:, _D * j:_D * (j + 1)]
            rj = r_ref[0, j, :]
            g = lax.shift_right_logical(rj, 1)

            diff_t = jnp.dot(dj, tmat, preferred_element_type=jnp.float32)
            sel = (gcol == g[:, None])
            x = jnp.where(sel, diff_t, 0.0)
            out = jnp.dot(x, p_ref[...], preferred_element_type=jnp.float32)

            onehot_r = (rcol == rj[:, None]).astype(jnp.float32)
            r_e = jnp.dot(onehot_r, e_ref[...],
                          preferred_element_type=jnp.float32)

            s = out + r_e
            e_glob = 4 * (i * _BR + krow) + j
            s = jnp.where(e_glob < _E, s, 0.0)
            part = part + jnp.sum(s * s)

        @pl.when(i == 0)
        def _init():
            o_ref[...] = jnp.zeros((1, 1), jnp.float32)

        o_ref[...] = o_ref[...] + part

        @pl.when(i == _GB - 1)
        def _final():
            o_ref[...] = o_ref[...] * (1.0 / (_E * _D))

    return pl.pallas_call(
        body,
        grid=(_GB,),
        in_specs=[
            pl.BlockSpec((_BR, 128), lambda i: (i, 0)),
            pl.BlockSpec((1, 8, _BR), lambda i: (i, 0, 0)),
            pl.BlockSpec((_D * _D, _D), lambda i: (0, 0)),
            pl.BlockSpec((64, _D), lambda i: (0, 0)),
        ],
        out_specs=pl.BlockSpec((1, 1), lambda i: (0, 0)),
        out_shape=jax.ShapeDtypeStruct((1, 1), jnp.float32),
    )(diff2d, ridx3, p_stacked, r_emb_w)


def kernel(node_emb, r_emb_w, r_proj_w, edge_index_t, edge_attr):
    pad = _EPAD - _E
    head_idx = jnp.concatenate(
        [edge_index_t[:, 0], jnp.zeros((pad,), jnp.int32)]).astype(jnp.int32)
    tail_idx = jnp.concatenate(
        [edge_index_t[:, 1], jnp.zeros((pad,), jnp.int32)]).astype(jnp.int32)
    head_idx = head_idx.reshape(_NW, _CH, _CHUNK)
    tail_idx = tail_idx.reshape(_NW, _CH, _CHUNK)

    diff_rows = _sc_gather(node_emb, head_idx, tail_idx)

    ridx = jnp.concatenate(
        [edge_attr[:, 0], jnp.zeros((pad,), jnp.int32)]).astype(jnp.int32)
    ridx3 = ridx.reshape(_GB, _BT // 4, 4).transpose(0, 2, 1)
    ridx3 = jnp.pad(ridx3, ((0, 0), (0, 4), (0, 0)))

    p_stacked = r_proj_w.reshape(_D * _D, _D)

    loss = _tc_loss(diff_rows.reshape(_EPAD // 4, 128),
                    ridx3, p_stacked, r_emb_w)
    return loss[0, 0]
